# baseline (speedup 1.0000x reference)
import functools

import jax
import jax.numpy as jnp
from jax import lax
from jax.experimental import pallas as pl
from jax.experimental.pallas import tpu as pltpu
from jax.experimental.pallas import tpu_sc as plsc

L = 16
NC, NS = 2, 16
NW = NC * NS
BATCH = 256
H, W = 513, 257
HS = 16
NSTRIPE = H // HS
WV = W // L
NSH = 257
NSP = 272
RPW = BATCH // NW
EPS = 1e-5


def _row_vecs(x_buf, w_buf, idx_buf, acc, hh, tail_mask):
    fo = hh * W
    for v in range(WV):
        o = v * L
        wv = w_buf[pl.ds(fo + o, L)]
        iv = idx_buf[pl.ds(fo + o, L)]
        for r in range(RPW):
            xv = x_buf[r, hh, pl.ds(o, L)]
            yv = xv * xv * wv
            plsc.addupdate_scatter(acc, [iv + (r * NSP)], yv)
    o = W - L
    wv = w_buf[pl.ds(fo + o, L)]
    iv = idx_buf[pl.ds(fo + o, L)]
    for r in range(RPW):
        xv = x_buf[r, hh, pl.ds(o, L)]
        yv = xv * xv * wv
        plsc.addupdate_scatter(acc, [iv + (r * NSP)], yv, mask=tail_mask)


def _body(x_hbm, w_hbm, idx_hbm, cnt_hbm, out_hbm,
          x_buf, w_buf, idx_buf, xr_buf, wr_buf, ir_buf,
          acc, cnt_buf, rec, out_buf):
    wid = lax.axis_index("s") * NC + lax.axis_index("c")
    row0 = wid * RPW
    tail_mask = lax.iota(jnp.int32, L) == (L - 1)

    zeros = jnp.zeros((L,), jnp.float32)

    def zbody(i, c):
        acc[pl.ds(i * L, L)] = zeros
        return c

    lax.fori_loop(0, (RPW * NSP) // L, zbody, 0)

    def sbody(s, carry):
        h0 = pl.multiple_of(s * HS, HS)
        p0 = pl.multiple_of(s * (HS * W), HS * W)
        pltpu.sync_copy(x_hbm.at[pl.ds(row0, RPW), 0, pl.ds(h0, HS)], x_buf)
        pltpu.sync_copy(w_hbm.at[pl.ds(p0, HS * W)], w_buf)
        pltpu.sync_copy(idx_hbm.at[pl.ds(p0, HS * W)], idx_buf)

        def hbody(hh, c):
            _row_vecs(x_buf, w_buf, idx_buf, acc, hh, tail_mask)
            return c

        lax.fori_loop(0, HS, hbody, 0)
        return carry

    lax.fori_loop(0, NSTRIPE, sbody, 0)

    pltpu.sync_copy(x_hbm.at[pl.ds(row0, RPW), 0, pl.ds(H - 1, 1)], xr_buf)
    pltpu.sync_copy(w_hbm.at[pl.ds((H - 1) * W, W)], wr_buf)
    pltpu.sync_copy(idx_hbm.at[pl.ds((H - 1) * W, W)], ir_buf)
    _row_vecs(xr_buf, wr_buf, ir_buf, acc, 0, tail_mask)

    pltpu.sync_copy(cnt_hbm, cnt_buf)
    for v in range(NSP // L):
        o = v * L
        rec[pl.ds(o, L)] = 1.0 / (cnt_buf[pl.ds(o, L)] + EPS)
    for r in range(RPW):
        for v in range(NSP // L):
            o = v * L
            out_buf[r, pl.ds(o, L)] = acc[pl.ds(r * NSP + o, L)] * rec[pl.ds(o, L)]
    pltpu.sync_copy(out_buf, out_hbm.at[pl.ds(row0, RPW)])


@jax.jit
def _sc_spectrum(x4, wf, idxf, cnt):
    mesh = plsc.VectorSubcoreMesh(core_axis_name="c", subcore_axis_name="s")
    f = pl.kernel(
        _body,
        mesh=mesh,
        compiler_params=pltpu.CompilerParams(
            needs_layout_passes=False, use_tc_tiling_on_sc=False),
        out_type=jax.ShapeDtypeStruct((BATCH, NSP), jnp.float32),
        scratch_types=[
            pltpu.VMEM((RPW, HS, W), jnp.float32),
            pltpu.VMEM((HS * W,), jnp.float32),
            pltpu.VMEM((HS * W,), jnp.int32),
            pltpu.VMEM((RPW, 1, W), jnp.float32),
            pltpu.VMEM((W,), jnp.float32),
            pltpu.VMEM((W,), jnp.int32),
            pltpu.VMEM((RPW * NSP,), jnp.float32),
            pltpu.VMEM((NSP,), jnp.float32),
            pltpu.VMEM((NSP,), jnp.float32),
            pltpu.VMEM((RPW, NSP), jnp.float32),
        ],
    )
    return f(x4, wf, idxf, cnt)


def kernel(x, shells_weight, shell_index, shells_count):
    b, c, h, w_ = x.shape
    wf = shells_weight.reshape(-1)
    idxf = shell_index.reshape(-1)
    cnt = jnp.concatenate(
        [shells_count, jnp.ones((NSP - NSH,), jnp.float32)])
    out = _sc_spectrum(x, wf, idxf, cnt)
    return out[:, :NSH].reshape(b, c, NSH)


# speedup vs baseline: 1.0489x; 1.0489x over previous
import functools

import jax
import jax.numpy as jnp
from jax import lax
from jax.experimental import pallas as pl
from jax.experimental.pallas import tpu as pltpu
from jax.experimental.pallas import tpu_sc as plsc

L = 16
NC, NS = 2, 16
NW = NC * NS
BATCH = 256
NPX = 513 * 257
PC = 4096
PCV = PC // L
NFULL = (NPX - 1) // PC
REM = (NPX - 1) - NFULL * PC
REM_VECS = REM // L
NSH = 257
NSP = 272
RPW = BATCH // NW
EPS = 1e-5


def _body(x_hbm, w_hbm, idx_hbm, cnt_hbm, out_hbm,
          x_buf, w_buf, idx_buf,
          a0, a1, a2, a3, a4, a5, a6, a7,
          cnt_buf, rec, out_buf, sem):
    accs = (a0, a1, a2, a3, a4, a5, a6, a7)
    wid = lax.axis_index("s") * NC + lax.axis_index("c")
    row0 = wid * RPW

    zeros = jnp.zeros((L,), jnp.float32)

    def zbody(i, c):
        o = i * L
        for r in range(RPW):
            accs[r][pl.ds(o, L)] = zeros
        return c

    lax.fori_loop(0, NSP // L, zbody, 0)

    def chunk_dmas(c, slot):
        base = pl.multiple_of(c * PC, PC)
        return (
            pltpu.make_async_copy(
                x_hbm.at[pl.ds(row0, RPW), pl.ds(base, PC)], x_buf.at[slot],
                sem),
            pltpu.make_async_copy(w_hbm.at[pl.ds(base, PC)], w_buf.at[slot],
                                  sem),
            pltpu.make_async_copy(idx_hbm.at[pl.ds(base, PC)],
                                  idx_buf.at[slot], sem),
        )

    def start(c, slot):
        for d in chunk_dmas(c, slot):
            d.start()

    def wait(c, slot):
        for d in chunk_dmas(c, slot):
            d.wait()

    def compute(slot, nvec):
        @plsc.parallel_loop(0, nvec, unroll=2)
        def vbody(i):
            o = i * L
            wv = w_buf[slot, pl.ds(o, L)]
            iv = idx_buf[slot, pl.ds(o, L)]
            for r in range(RPW):
                xv = x_buf[slot, r, pl.ds(o, L)]
                yv = xv * xv * wv
                plsc.addupdate_scatter(accs[r], [iv], yv)

    start(0, 0)

    def cbody(c, carry):
        slot = lax.rem(c, 2)
        wait(c, slot)

        @pl.when(c + 1 < NFULL)
        def _():
            start(c + 1, 1 - slot)

        compute(slot, PCV)
        return carry

    lax.fori_loop(0, NFULL, cbody, 0)

    rbase = NFULL * PC
    pltpu.sync_copy(x_hbm.at[pl.ds(row0, RPW), pl.ds(rbase, REM)],
                    x_buf.at[0, :, pl.ds(0, REM)])
    pltpu.sync_copy(w_hbm.at[pl.ds(rbase, REM)], w_buf.at[0, pl.ds(0, REM)])
    pltpu.sync_copy(idx_hbm.at[pl.ds(rbase, REM)],
                    idx_buf.at[0, pl.ds(0, REM)])
    compute(0, REM_VECS)

    pltpu.sync_copy(cnt_hbm, cnt_buf)
    for v in range(NSP // L):
        o = v * L
        rec[pl.ds(o, L)] = 1.0 / (cnt_buf[pl.ds(o, L)] + EPS)
    for r in range(RPW):
        for v in range(NSP // L):
            o = v * L
            out_buf[r, pl.ds(o, L)] = accs[r][pl.ds(o, L)] * rec[pl.ds(o, L)]
    pltpu.sync_copy(out_buf, out_hbm.at[pl.ds(row0, RPW)])


@jax.jit
def _sc_spectrum(x2, w, idx, cnt):
    mesh = plsc.VectorSubcoreMesh(core_axis_name="c", subcore_axis_name="s")
    f = pl.kernel(
        _body,
        mesh=mesh,
        compiler_params=pltpu.CompilerParams(
            needs_layout_passes=False, use_tc_tiling_on_sc=False),
        out_type=jax.ShapeDtypeStruct((BATCH, NSP), jnp.float32),
        scratch_types=(
            [
                pltpu.VMEM((2, RPW, PC), jnp.float32),
                pltpu.VMEM((2, PC), jnp.float32),
                pltpu.VMEM((2, PC), jnp.int32),
            ]
            + [pltpu.VMEM((NSP,), jnp.float32) for _ in range(RPW)]
            + [
                pltpu.VMEM((NSP,), jnp.float32),
                pltpu.VMEM((NSP,), jnp.float32),
                pltpu.VMEM((RPW, NSP), jnp.float32),
                pltpu.SemaphoreType.DMA,
            ]
        ),
    )
    return f(x2, w, idx, cnt)


def kernel(x, shells_weight, shell_index, shells_count):
    b, c, h, w_ = x.shape
    x2 = x.reshape(b, h * w_)
    wf = shells_weight.reshape(-1)
    idxf = shell_index.reshape(-1)
    cnt = jnp.concatenate(
        [shells_count, jnp.ones((NSP - NSH,), jnp.float32)])
    out = _sc_spectrum(x2, wf, idxf, cnt)
    out = out[:, :NSH]
    last = x2[:, NPX - 1]
    contrib = (last * last) * wf[NPX - 1] / (shells_count[idxf[NPX - 1]] + EPS)
    out = out.at[:, idxf[NPX - 1]].add(contrib)
    return out.reshape(b, c, NSH)


# speedup vs baseline: 3.1161x; 2.9707x over previous
import functools

import jax
import jax.numpy as jnp
from jax import lax
from jax.experimental import pallas as pl
from jax.experimental.pallas import tpu as pltpu
from jax.experimental.pallas import tpu_sc as plsc

L = 16
NC, NS = 2, 16
NW = NC * NS
BATCH = 256
NPX = 513 * 257
PC = 4096
PCV = PC // L
NFULL = (NPX - 1) // PC
REM = (NPX - 1) - NFULL * PC
REM_VECS = REM // L
NSH = 257
NSP = 272
RPW = BATCH // NW
EPS = 1e-5


def _body(x_hbm, w_hbm, idx_hbm, cnt_hbm, out_hbm,
          x_buf, w_buf, idx_buf,
          a0, a1, a2, a3, a4, a5, a6, a7,
          cnt_buf, rec, out_buf, sem):
    accs = (a0, a1, a2, a3, a4, a5, a6, a7)
    wid = lax.axis_index("s") * NC + lax.axis_index("c")
    row0 = wid * RPW

    zeros = jnp.zeros((L,), jnp.float32)

    def zbody(i, c):
        o = i * L
        for r in range(RPW):
            accs[r][pl.ds(o, L)] = zeros
        return c

    lax.fori_loop(0, NSP // L, zbody, 0)

    def chunk_dmas(c, slot):
        base = pl.multiple_of(c * PC, PC)
        return (
            pltpu.make_async_copy(
                x_hbm.at[pl.ds(row0, RPW), pl.ds(base, PC)], x_buf.at[slot],
                sem),
            pltpu.make_async_copy(w_hbm.at[pl.ds(base, PC)], w_buf.at[slot],
                                  sem),
            pltpu.make_async_copy(idx_hbm.at[pl.ds(base, PC)],
                                  idx_buf.at[slot], sem),
        )

    def start(c, slot):
        for d in chunk_dmas(c, slot):
            d.start()

    def wait(c, slot):
        for d in chunk_dmas(c, slot):
            d.wait()

    def compute(slot, nvec):
        @plsc.parallel_loop(0, nvec, unroll=2)
        def vbody(i):
            o = i * L
            wv = w_buf[slot, pl.ds(o, L)]
            iv = idx_buf[slot, pl.ds(o, L)]
            for r in range(RPW):
                xv = x_buf[slot, r, pl.ds(o, L)]
                yv = xv * xv * wv
                plsc.addupdate_scatter(accs[r], [iv], yv)

    start(0, 0)

    def cbody(c, carry):
        slot = lax.rem(c, 2)
        wait(c, slot)

        @pl.when(c + 1 < NFULL)
        def _():
            start(c + 1, 1 - slot)

        compute(slot, PCV)
        return carry

    lax.fori_loop(0, NFULL, cbody, 0)

    rbase = NFULL * PC
    pltpu.sync_copy(x_hbm.at[pl.ds(row0, RPW), pl.ds(rbase, REM)],
                    x_buf.at[0, :, pl.ds(0, REM)])
    pltpu.sync_copy(w_hbm.at[pl.ds(rbase, REM)], w_buf.at[0, pl.ds(0, REM)])
    pltpu.sync_copy(idx_hbm.at[pl.ds(rbase, REM)],
                    idx_buf.at[0, pl.ds(0, REM)])
    compute(0, REM_VECS)

    pltpu.sync_copy(cnt_hbm, cnt_buf)
    for v in range(NSP // L):
        o = v * L
        rec[pl.ds(o, L)] = 1.0 / (cnt_buf[pl.ds(o, L)] + EPS)
    for r in range(RPW):
        for v in range(NSP // L):
            o = v * L
            out_buf[r, pl.ds(o, L)] = accs[r][pl.ds(o, L)] * rec[pl.ds(o, L)]
    pltpu.sync_copy(out_buf, out_hbm.at[pl.ds(row0, RPW)])


@jax.jit
def _sc_spectrum(x2, w, idx, cnt):
    mesh = plsc.VectorSubcoreMesh(core_axis_name="c", subcore_axis_name="s")
    f = pl.kernel(
        _body,
        mesh=mesh,
        compiler_params=pltpu.CompilerParams(needs_layout_passes=False),
        out_type=jax.ShapeDtypeStruct((BATCH, NSP), jnp.float32),
        scratch_types=(
            [
                pltpu.VMEM((2, RPW, PC), jnp.float32),
                pltpu.VMEM((2, PC), jnp.float32),
                pltpu.VMEM((2, PC), jnp.int32),
            ]
            + [pltpu.VMEM((NSP,), jnp.float32) for _ in range(RPW)]
            + [
                pltpu.VMEM((NSP,), jnp.float32),
                pltpu.VMEM((NSP,), jnp.float32),
                pltpu.VMEM((RPW, NSP), jnp.float32),
                pltpu.SemaphoreType.DMA,
            ]
        ),
    )
    return f(x2, w, idx, cnt)


def kernel(x, shells_weight, shell_index, shells_count):
    b, c, h, w_ = x.shape
    x2 = x.reshape(b, h * w_)
    wf = shells_weight.reshape(-1)
    idxf = shell_index.reshape(-1)
    cnt = jnp.concatenate(
        [shells_count, jnp.ones((NSP - NSH,), jnp.float32)])
    out = _sc_spectrum(x2, wf, idxf, cnt)
    out = out[:, :NSH]
    last = x2[:, NPX - 1]
    contrib = (last * last) * wf[NPX - 1] / (shells_count[idxf[NPX - 1]] + EPS)
    out = out.at[:, idxf[NPX - 1]].add(contrib)
    return out.reshape(b, c, NSH)


# speedup vs baseline: 4.8629x; 1.5606x over previous
import functools

import jax
import jax.numpy as jnp
from jax import lax
from jax.experimental import pallas as pl
from jax.experimental.pallas import tpu as pltpu
from jax.experimental.pallas import tpu_sc as plsc

L = 16
NC, NS = 2, 16
NW = NC * NS
BATCH = 256
H, W = 513, 257
HM, WM = H - 1, W - 1
HS = 16
NSTRIPE = HM // HS
WV = WM // L
LP = W + HM
LPP = 784
LPV = LPP // L
NSH = 257
NSP = 272
RPW = BATCH // NW
EPS = 1e-5


def _body(x_hbm, w_hbm, idx_hbm, xl_hbm, wl_hbm, il_hbm, cnt_hbm, out_hbm,
          x_buf, w_buf, idx_buf, xl_buf, wl_buf, il_buf,
          a0, a1, a2, a3, a4, a5, a6, a7,
          cnt_buf, rec, out_buf, sem):
    accs = (a0, a1, a2, a3, a4, a5, a6, a7)
    wid = lax.axis_index("s") * NC + lax.axis_index("c")
    row0 = wid * RPW

    zeros = jnp.zeros((L,), jnp.float32)

    def zbody(i, c):
        o = i * L
        for r in range(RPW):
            accs[r][pl.ds(o, L)] = zeros
        return c

    lax.fori_loop(0, NSP // L, zbody, 0)

    def chunk_dmas(s, slot):
        h0 = pl.multiple_of(s * HS, HS)
        return (
            pltpu.make_async_copy(
                x_hbm.at[pl.ds(row0, RPW), pl.ds(h0, HS), pl.ds(0, WM)],
                x_buf.at[slot], sem),
            pltpu.make_async_copy(
                w_hbm.at[pl.ds(h0, HS), pl.ds(0, WM)], w_buf.at[slot], sem),
            pltpu.make_async_copy(
                idx_hbm.at[pl.ds(h0, HS), pl.ds(0, WM)], idx_buf.at[slot],
                sem),
        )

    def start(s, slot):
        for d in chunk_dmas(s, slot):
            d.start()

    def wait(s, slot):
        for d in chunk_dmas(s, slot):
            d.wait()

    def compute(slot):
        @plsc.parallel_loop(0, HS)
        def hbody(hh):
            for v in range(WV):
                o = v * L
                wv = w_buf[slot, hh, pl.ds(o, L)]
                iv = idx_buf[slot, hh, pl.ds(o, L)]
                for r in range(RPW):
                    xv = x_buf[slot, r, hh, pl.ds(o, L)]
                    yv = xv * xv * wv
                    plsc.addupdate_scatter(accs[r], [iv], yv)

    start(0, 0)
    pltpu.sync_copy(xl_hbm.at[pl.ds(row0, RPW)], xl_buf)
    pltpu.sync_copy(wl_hbm, wl_buf)
    pltpu.sync_copy(il_hbm, il_buf)

    def cbody(s, carry):
        slot = lax.rem(s, 2)
        wait(s, slot)

        @pl.when(s + 1 < NSTRIPE)
        def _():
            start(s + 1, 1 - slot)

        compute(slot)
        return carry

    lax.fori_loop(0, NSTRIPE, cbody, 0)

    @plsc.parallel_loop(0, LPV)
    def lbody(i):
        o = i * L
        wv = wl_buf[pl.ds(o, L)]
        iv = il_buf[pl.ds(o, L)]
        for r in range(RPW):
            xv = xl_buf[r, pl.ds(o, L)]
            yv = xv * xv * wv
            plsc.addupdate_scatter(accs[r], [iv], yv)

    pltpu.sync_copy(cnt_hbm, cnt_buf)
    for v in range(NSP // L):
        o = v * L
        rec[pl.ds(o, L)] = 1.0 / (cnt_buf[pl.ds(o, L)] + EPS)
    for r in range(RPW):
        for v in range(NSP // L):
            o = v * L
            out_buf[r, pl.ds(o, L)] = accs[r][pl.ds(o, L)] * rec[pl.ds(o, L)]
    pltpu.sync_copy(out_buf, out_hbm.at[pl.ds(row0, RPW)])


@jax.jit
def _sc_spectrum(x3, w2, idx2, xl, wl, il, cnt):
    mesh = plsc.VectorSubcoreMesh(core_axis_name="c", subcore_axis_name="s")
    f = pl.kernel(
        _body,
        mesh=mesh,
        compiler_params=pltpu.CompilerParams(needs_layout_passes=False),
        out_type=jax.ShapeDtypeStruct((BATCH, NSP), jnp.float32),
        scratch_types=(
            [
                pltpu.VMEM((2, RPW, HS, WM), jnp.float32),
                pltpu.VMEM((2, HS, WM), jnp.float32),
                pltpu.VMEM((2, HS, WM), jnp.int32),
                pltpu.VMEM((RPW, LPP), jnp.float32),
                pltpu.VMEM((LPP,), jnp.float32),
                pltpu.VMEM((LPP,), jnp.int32),
            ]
            + [pltpu.VMEM((NSP,), jnp.float32) for _ in range(RPW)]
            + [
                pltpu.VMEM((NSP,), jnp.float32),
                pltpu.VMEM((NSP,), jnp.float32),
                pltpu.VMEM((RPW, NSP), jnp.float32),
                pltpu.SemaphoreType.DMA,
            ]
        ),
    )
    return f(x3, w2, idx2, xl, wl, il, cnt)


def kernel(x, shells_weight, shell_index, shells_count):
    b, c, h, w_ = x.shape
    x3 = x.reshape(b, h, w_)
    xl = jnp.concatenate([x3[:, h - 1, :], x3[:, : h - 1, w_ - 1]], axis=1)
    xl = jnp.pad(xl, ((0, 0), (0, LPP - LP)))
    wl = jnp.concatenate(
        [shells_weight[h - 1, :], shells_weight[: h - 1, w_ - 1],
         jnp.zeros((LPP - LP,), jnp.float32)])
    il = jnp.concatenate(
        [shell_index[h - 1, :], shell_index[: h - 1, w_ - 1],
         jnp.zeros((LPP - LP,), jnp.int32)])
    cnt = jnp.concatenate(
        [shells_count, jnp.ones((NSP - NSH,), jnp.float32)])
    out = _sc_spectrum(x3, shells_weight, shell_index, xl, wl, il, cnt)
    return out[:, :NSH].reshape(b, c, NSH)
